# Initial kernel scaffold; baseline (speedup 1.0000x reference)
import jax
import jax.numpy as jnp
from jax.experimental import pallas as pl


def kernel(h, params, edge_index):
    raise NotImplementedError("write your pallas kernel here")


# speedup vs baseline: 1.9792x; 1.9792x over previous
import functools

import jax
import jax.numpy as jnp
from jax import lax
from jax.experimental import pallas as pl
from jax.experimental.pallas import tpu as pltpu
from jax.experimental.pallas import tpu_sc as plsc

N = 100000
E = 1600000
DIN = 17
H = 64
H2 = 128
OUT = 2
NEG = 0.05
EPS = 1e-5

NC = 2
NS = 16
NW = NC * NS
L = 16

SHIFT = 9
CH = 1 << SHIFT
C = (N + CH - 1) // CH
NPAD = C * CH
CB = 256
DEAD = NPAD

E_PAD = 1605632
ROWS = E_PAD // 128
ROWS_T16 = ROWS // 16
ROWS_T32 = ROWS // 32
PERM_PAD = E_PAD + 256
ACC1 = 102400
W_E = 256

BN = 2000
GRID = N // BN

_i32 = jnp.int32
_f32 = jnp.float32


def _mesh():
    return plsc.VectorSubcoreMesh(core_axis_name="c", subcore_axis_name="s")


def _wid():
    return lax.axis_index("s") * NC + lax.axis_index("c")


def _iota():
    return lax.iota(_i32, L)


def _k1_body(src2d, dst2d, deg_out, deg_in, acc, idxbuf, onesbuf, zbuf):
    cid = lax.axis_index("c")
    sid = lax.axis_index("s")

    def zb(j, _):
        zbuf[pl.ds(j * L, L)] = jnp.zeros((L,), _f32)
        return 0
    lax.fori_loop(0, 6400 // L, zb, 0)
    pltpu.sync_copy(zbuf, acc.at[pl.ds(sid * 6400, 6400)])

    for j in range(128 // L):
        onesbuf[0, pl.ds(j * L, L)] = jnp.ones((L,), _f32)
    plsc.subcore_barrier()

    def accum(ref2d):
        base_row = sid * ROWS_T16

        def win_body(w, _):
            pltpu.sync_copy(ref2d.at[pl.ds(base_row + w * 16, 16)], idxbuf)
            for k in range(16):
                pltpu.sync_copy(onesbuf.at[0], acc.at[idxbuf.at[k]], add=True)
            return 0
        lax.fori_loop(0, ROWS_T16 // 16, win_body, 0)
        rem = ROWS_T16 % 16
        if rem:
            pltpu.sync_copy(ref2d.at[pl.ds(base_row + ROWS_T16 - rem, rem)],
                            idxbuf.at[pl.ds(0, rem)])
            for k in range(rem):
                pltpu.sync_copy(onesbuf.at[0], acc.at[idxbuf.at[k]], add=True)

    @pl.when(cid == 0)
    def _():
        accum(src2d)

    @pl.when(cid == 1)
    def _():
        accum(dst2d)

    plsc.subcore_barrier()

    def copy_out(dst_ref):
        n_per = 6400
        @pl.when(sid < NS - 1)
        def _():
            pltpu.sync_copy(acc.at[pl.ds(sid * n_per, n_per)], zbuf)
            pltpu.sync_copy(zbuf, dst_ref.at[pl.ds(sid * n_per, n_per)])
        last = N - (NS - 1) * n_per
        @pl.when(sid == NS - 1)
        def _():
            pltpu.sync_copy(acc.at[pl.ds((NS - 1) * n_per, last)],
                            zbuf.at[pl.ds(0, last)])
            pltpu.sync_copy(zbuf.at[pl.ds(0, last)],
                            dst_ref.at[pl.ds((NS - 1) * n_per, last)])

    @pl.when(cid == 0)
    def _():
        copy_out(deg_out)

    @pl.when(cid == 1)
    def _():
        copy_out(deg_in)


@functools.lru_cache(maxsize=None)
def _k1():
    return functools.partial(
        pl.kernel,
        mesh=_mesh(),
        compiler_params=pltpu.CompilerParams(needs_layout_passes=False),
        out_type=(jax.ShapeDtypeStruct((N,), _f32),
                  jax.ShapeDtypeStruct((N,), _f32)),
        scratch_types=[
            pltpu.VMEM_SHARED((ACC1,), _f32),
            pltpu.VMEM((16, 128), _i32),
            pltpu.VMEM((1, 128), _f32),
            pltpu.VMEM((6400,), _f32),
        ],
    )(_k1_body)


def _k2a_body(dst2d, hists, win, hist2, histv):
    wid = _wid()
    iota = _iota()
    lanebase = iota * CB
    ones = jnp.ones((L,), _i32)

    def zb(j, _):
        hist2[pl.ds(j * L, L)] = jnp.zeros((L,), _i32)
        return 0
    lax.fori_loop(0, (L * CB) // L, zb, 0)

    base_row = wid * ROWS_T32

    def hrow(k):
        for v in range(8):
            d16 = win[k, pl.ds(v * L, L)]
            b16 = lax.shift_right_logical(d16, SHIFT)
            plsc.addupdate_scatter(hist2, [lanebase + b16], ones)

    def win_body(w, _):
        pltpu.sync_copy(dst2d.at[pl.ds(base_row + w * 16, 16)], win)

        def row_body(k, _):
            hrow(k)
            return 0
        lax.fori_loop(0, 16, row_body, 0)
        return 0
    lax.fori_loop(0, ROWS_T32 // 16, win_body, 0)
    rem = ROWS_T32 % 16
    if rem:
        pltpu.sync_copy(dst2d.at[pl.ds(base_row + ROWS_T32 - rem, rem)],
                        win.at[pl.ds(0, rem)])
        def row_body_r(k, _):
            hrow(k)
            return 0
        lax.fori_loop(0, rem, row_body_r, 0)

    def red(g, _):
        s = hist2[pl.ds(g * L, L)]
        for r in range(1, 16):
            s = s + hist2[pl.ds(r * CB + g * L, L)]
        histv[pl.ds(g * L, L)] = s
        return 0
    lax.fori_loop(0, CB // L, red, 0)
    pltpu.sync_copy(histv, hists.at[pl.ds(wid * CB, CB)])


@functools.lru_cache(maxsize=None)
def _k2a():
    return functools.partial(
        pl.kernel,
        mesh=_mesh(),
        compiler_params=pltpu.CompilerParams(needs_layout_passes=False),
        out_type=jax.ShapeDtypeStruct((NW * CB,), _i32),
        scratch_types=[
            pltpu.VMEM((16, 128), _i32),
            pltpu.VMEM((L * CB,), _i32),
            pltpu.VMEM((CB,), _i32),
        ],
    )(_k2a_body)


def _k2b_body(dst2d, src2d, hists, perm_src, perm_dst, offsets,
              histsv, offs_all, cursors, dwin, swin,
              kbt, dt, st, posbuf, sbuf, dbuf, offv, ztail):
    wid = _wid()
    iota = _iota()

    pltpu.sync_copy(hists, histsv)

    def scan_body(j, carry):
        idx16 = j * L + iota
        b16 = lax.shift_right_logical(idx16, 5)
        t16 = idx16 & 31
        cnt = plsc.load_gather(histsv, [t16 * CB + b16])
        incl = plsc.cumsum(cnt)
        offs_all[pl.ds(j * L, L)] = incl - cnt + carry
        return carry + jnp.sum(cnt)
    lax.fori_loop(0, (CB * NW) // L, scan_body, jnp.int32(0))

    def cur_body(g, _):
        b16 = g * L + iota
        cursors[pl.ds(g * L, L)] = plsc.load_gather(offs_all, [b16 * NW + wid])
        return 0
    lax.fori_loop(0, CB // L, cur_body, 0)

    @pl.when(wid == 0)
    def _():
        def off_body(g, _):
            b16 = g * L + iota
            offv[pl.ds(g * L, L)] = plsc.load_gather(offs_all, [b16 * NW])
            return 0
        lax.fori_loop(0, CB // L, off_body, 0)
        pltpu.sync_copy(offv, offsets)

        def zt(j, _):
            ztail[pl.ds(j * L, L)] = jnp.zeros((L,), _i32)
            return 0
        lax.fori_loop(0, 256 // L, zt, 0)
        pltpu.sync_copy(ztail, perm_src.at[pl.ds(E_PAD, 256)])
        pltpu.sync_copy(ztail, perm_dst.at[pl.ds(E_PAD, 256)])

    base_row = wid * ROWS_T32

    def do_row(k):
        for v in range(8):
            d16 = dwin[k, pl.ds(v * L, L)]
            s16 = swin[k, pl.ds(v * L, L)]
            b16 = lax.shift_right_logical(d16, SHIFT)
            kb, vl = plsc.sort_key_val(b16, iota)
            kbt[...] = kb
            prev = plsc.load_gather(kbt, [jnp.maximum(iota - 1, 0)])
            nxt = plsc.load_gather(kbt, [jnp.minimum(iota + 1, L - 1)])
            isstart = (iota == 0) | (kb != prev)
            islast = (iota == L - 1) | (kb != nxt)
            runstart = plsc.cummax(jnp.where(isstart, iota, 0))
            rank = iota - runstart
            base = plsc.load_gather(cursors, [kb])
            pos = base + rank
            plsc.store_scatter(cursors, [kb], pos + 1, mask=islast)
            dt[...] = d16
            st[...] = s16
            dperm = plsc.load_gather(dt, [vl])
            sperm = plsc.load_gather(st, [vl])
            posbuf[0, pl.ds(v * L, L)] = pos
            sbuf[0, pl.ds(v * L, L)] = sperm
            dbuf[0, pl.ds(v * L, L)] = dperm
        pltpu.sync_copy(sbuf.at[0], perm_src.at[posbuf.at[0]])
        pltpu.sync_copy(dbuf.at[0], perm_dst.at[posbuf.at[0]])

    def win_body(w, _):
        pltpu.sync_copy(dst2d.at[pl.ds(base_row + w * 16, 16)], dwin)
        pltpu.sync_copy(src2d.at[pl.ds(base_row + w * 16, 16)], swin)

        def row_body(k, _):
            do_row(k)
            return 0
        lax.fori_loop(0, 16, row_body, 0)
        return 0
    lax.fori_loop(0, ROWS_T32 // 16, win_body, 0)

    rem = ROWS_T32 % 16
    if rem:
        pltpu.sync_copy(dst2d.at[pl.ds(base_row + ROWS_T32 - rem, rem)],
                        dwin.at[pl.ds(0, rem)])
        pltpu.sync_copy(src2d.at[pl.ds(base_row + ROWS_T32 - rem, rem)],
                        swin.at[pl.ds(0, rem)])

        def row_body_r(k, _):
            do_row(k)
            return 0
        lax.fori_loop(0, rem, row_body_r, 0)


@functools.lru_cache(maxsize=None)
def _k2b():
    return functools.partial(
        pl.kernel,
        mesh=_mesh(),
        compiler_params=pltpu.CompilerParams(needs_layout_passes=False),
        out_type=(jax.ShapeDtypeStruct((PERM_PAD,), _i32),
                  jax.ShapeDtypeStruct((PERM_PAD,), _i32),
                  jax.ShapeDtypeStruct((CB,), _i32)),
        scratch_types=[
            pltpu.VMEM((NW * CB,), _i32),
            pltpu.VMEM((CB * NW,), _i32),
            pltpu.VMEM((CB,), _i32),
            pltpu.VMEM((16, 128), _i32),
            pltpu.VMEM((16, 128), _i32),
            pltpu.VMEM((L,), _i32),
            pltpu.VMEM((L,), _i32),
            pltpu.VMEM((L,), _i32),
            pltpu.VMEM((1, 128), _i32),
            pltpu.VMEM((1, 128), _i32),
            pltpu.VMEM((1, 128), _i32),
            pltpu.VMEM((CB,), _i32),
            pltpu.VMEM((256,), _i32),
        ],
    )(_k2b_body)


def _make_seg_body(width, is_max):
    nj = width // L

    def body(feat, perm_src, perm_dst, offsets, out, acc, gbuf, swin, dwin,
             offv):
        wid = _wid()
        pltpu.sync_copy(offsets, offv)

        def run_chunk(c):
            def zb(i, _):
                for j in range(nj):
                    acc[i, pl.ds(j * L, L)] = jnp.zeros((L,), _f32)
                return 0
            lax.fori_loop(0, CH, zb, 0)

            off2 = offv[pl.ds(c, L)]
            start0 = off2[0]
            end0 = off2[1]
            astart = start0 & jnp.int32(-8)
            nwin = (end0 - astart + (W_E - 1)) // W_E
            cbase = c * CH

            def win_body(i, _):
                ws = pl.multiple_of(astart + i * W_E, 8)
                pltpu.sync_copy(perm_src.at[pl.ds(ws, W_E)],
                                swin.at[pl.ds(0, W_E)])
                pltpu.sync_copy(perm_dst.at[pl.ds(ws, W_E)],
                                dwin.at[pl.ds(0, W_E)])
                pltpu.sync_copy(feat.at[swin.at[pl.ds(0, W_E)]], gbuf)
                lo = jnp.maximum(start0 - ws, 0)
                hi = jnp.minimum(end0 - ws, W_E)

                def edge_body(e, _):
                    dloc = dwin[pl.ds(e, L)][0] - cbase
                    for j in range(nj):
                        cur = acc[dloc, pl.ds(j * L, L)]
                        val = gbuf[e, pl.ds(j * L, L)]
                        if is_max:
                            acc[dloc, pl.ds(j * L, L)] = jnp.maximum(cur, val)
                        else:
                            acc[dloc, pl.ds(j * L, L)] = cur + val
                    return 0
                lax.fori_loop(lo, hi, edge_body, 0)
                return 0
            lax.fori_loop(0, nwin, win_body, 0)

            pltpu.sync_copy(acc.at[pl.ds(0, CH)], out.at[pl.ds(cbase, CH)])

        for t in range(C // NW):
            run_chunk(wid + t * NW)
        remc = C % NW
        if remc:
            @pl.when(wid < remc)
            def _():
                run_chunk(wid + (C // NW) * NW)

    return body


@functools.lru_cache(maxsize=None)
def _seg_kernel(width, is_max):
    return functools.partial(
        pl.kernel,
        mesh=_mesh(),
        compiler_params=pltpu.CompilerParams(needs_layout_passes=False),
        out_type=jax.ShapeDtypeStruct((NPAD, width), _f32),
        scratch_types=[
            pltpu.VMEM((CH, width), _f32),
            pltpu.VMEM((W_E, H2), _f32),
            pltpu.VMEM((W_E + L,), _i32),
            pltpu.VMEM((W_E + L,), _i32),
            pltpu.VMEM((CB,), _i32),
        ],
    )(_make_seg_body(width, is_max))


def _ln(x, g, b):
    m = jnp.mean(x, axis=-1, keepdims=True)
    v = jnp.mean((x - m) * (x - m), axis=-1, keepdims=True)
    return (x - m) * lax.rsqrt(v + EPS) * g + b


def _tca_body(h_ref, deg_ref, w_in, b_in, w_t1, b_t1, w_t2, b_t2, w_g, b_g,
              x_out, xs_out, pool_out, m_ref, s_ref, p_ref):
    i = pl.program_id(0)
    hb = h_ref[...]
    x = jnp.dot(hb, w_in[...], preferred_element_type=_f32) + b_in[...]
    x = jnp.dot(x, w_t1[...], preferred_element_type=_f32) + b_t1[...]
    x = jnp.where(x >= 0, x, NEG * x)
    x = jnp.dot(x, w_t2[...], preferred_element_type=_f32) + b_t2[...]
    x_out[...] = x
    deg = deg_ref[...]
    xs = x * lax.rsqrt(jnp.maximum(deg, 1.0))
    xs_out[...] = jnp.concatenate([xs, jnp.zeros_like(xs)], axis=1)

    g = jnp.dot(x, w_g[...], preferred_element_type=_f32) + b_g[...]

    @pl.when(i == 0)
    def _():
        m_ref[0] = -jnp.inf
        s_ref[0] = 0.0
        p_ref[...] = jnp.zeros_like(p_ref)

    bm = jnp.max(g)
    mo = m_ref[0]
    mn = jnp.maximum(mo, bm)
    corr = jnp.exp(mo - mn)
    e = jnp.exp(g - mn)
    s_new = s_ref[0] * corr + jnp.sum(e)
    s_ref[0] = s_new
    p_new = p_ref[...] * corr + jnp.sum(e * x, axis=0, keepdims=True)
    p_ref[...] = p_new
    pool_out[...] = p_new / s_new


@functools.lru_cache(maxsize=None)
def _tca():
    bs = pl.BlockSpec
    return pl.pallas_call(
        _tca_body,
        grid=(GRID,),
        in_specs=[
            bs((BN, DIN), lambda i: (i, 0)),
            bs((BN, 1), lambda i: (i, 0)),
            bs((DIN, H), lambda i: (0, 0)),
            bs((1, H), lambda i: (0, 0)),
            bs((H, H), lambda i: (0, 0)),
            bs((1, H), lambda i: (0, 0)),
            bs((H, H), lambda i: (0, 0)),
            bs((1, H), lambda i: (0, 0)),
            bs((H, 1), lambda i: (0, 0)),
            bs((1, 1), lambda i: (0, 0)),
        ],
        out_specs=[
            bs((BN, H), lambda i: (i, 0)),
            bs((BN, H2), lambda i: (i, 0)),
            bs((1, H), lambda i: (0, 0)),
        ],
        out_shape=[
            jax.ShapeDtypeStruct((N, H), _f32),
            jax.ShapeDtypeStruct((N, H2), _f32),
            jax.ShapeDtypeStruct((1, H), _f32),
        ],
        scratch_shapes=[
            pltpu.SMEM((1,), _f32),
            pltpu.SMEM((1,), _f32),
            pltpu.VMEM((1, H), _f32),
        ],
        compiler_params=pltpu.CompilerParams(
            dimension_semantics=("arbitrary",)),
    )


def _tcb_body(agg_ref, deg_ref, x_ref, pool_ref, w_gcn, b_gcn, g_gcn, bn_gcn,
              wp1a, wp1b, bp1, z_out, hp1_out):
    a = agg_ref[...] * lax.rsqrt(jnp.maximum(deg_ref[...], 1.0))
    t = jnp.dot(a, w_gcn[...], preferred_element_type=_f32) + b_gcn[...]
    g1 = _ln(t, g_gcn[...], bn_gcn[...])
    xb = x_ref[...]
    loc = g1 - xb
    glo = pool_ref[...] - xb
    z_out[...] = jnp.concatenate([loc, glo], axis=1)
    hp = (jnp.dot(loc, wp1a[...], preferred_element_type=_f32)
          + jnp.dot(glo, wp1b[...], preferred_element_type=_f32) + bp1[...])
    hp1_out[...] = jnp.maximum(hp, 0.0)


@functools.lru_cache(maxsize=None)
def _tcb():
    bs = pl.BlockSpec
    return pl.pallas_call(
        _tcb_body,
        grid=(GRID,),
        in_specs=[
            bs((BN, H), lambda i: (i, 0)),
            bs((BN, 1), lambda i: (i, 0)),
            bs((BN, H), lambda i: (i, 0)),
            bs((1, H), lambda i: (0, 0)),
            bs((H, H), lambda i: (0, 0)),
            bs((1, H), lambda i: (0, 0)),
            bs((1, H), lambda i: (0, 0)),
            bs((1, H), lambda i: (0, 0)),
            bs((H, H2), lambda i: (0, 0)),
            bs((H, H2), lambda i: (0, 0)),
            bs((1, H2), lambda i: (0, 0)),
        ],
        out_specs=[
            bs((BN, H2), lambda i: (i, 0)),
            bs((BN, H2), lambda i: (i, 0)),
        ],
        out_shape=[
            jax.ShapeDtypeStruct((N, H2), _f32),
            jax.ShapeDtypeStruct((N, H2), _f32),
        ],
        compiler_params=pltpu.CompilerParams(
            dimension_semantics=("arbitrary",)),
    )


def _tcc1_body(hh_ref, n_ref, ws, wn, bsb, g_ln, b_ln, wp2, bp2,
               hh1_out, hp2_out):
    o = (jnp.dot(hh_ref[...], ws[...], preferred_element_type=_f32)
         + jnp.dot(n_ref[...], wn[...], preferred_element_type=_f32)
         + bsb[...])
    r = jnp.maximum(_ln(o, g_ln[...], b_ln[...]), 0.0)
    hh1_out[...] = r
    hp = jnp.dot(r, wp2[...], preferred_element_type=_f32) + bp2[...]
    hp2_out[...] = jnp.maximum(hp, 0.0)


@functools.lru_cache(maxsize=None)
def _tcc1():
    bs = pl.BlockSpec
    return pl.pallas_call(
        _tcc1_body,
        grid=(GRID,),
        in_specs=[
            bs((BN, H2), lambda i: (i, 0)),
            bs((BN, H2), lambda i: (i, 0)),
            bs((H2, H2), lambda i: (0, 0)),
            bs((H2, H2), lambda i: (0, 0)),
            bs((1, H2), lambda i: (0, 0)),
            bs((1, H2), lambda i: (0, 0)),
            bs((1, H2), lambda i: (0, 0)),
            bs((H2, H2), lambda i: (0, 0)),
            bs((1, H2), lambda i: (0, 0)),
        ],
        out_specs=[
            bs((BN, H2), lambda i: (i, 0)),
            bs((BN, H2), lambda i: (i, 0)),
        ],
        out_shape=[
            jax.ShapeDtypeStruct((N, H2), _f32),
            jax.ShapeDtypeStruct((N, H2), _f32),
        ],
        compiler_params=pltpu.CompilerParams(
            dimension_semantics=("arbitrary",)),
    )


def _tcc2_body(hh_ref, n_ref, ws, wn, bsb, g_ln, b_ln, wo, bo, score_out):
    o = (jnp.dot(hh_ref[...], ws[...], preferred_element_type=_f32)
         + jnp.dot(n_ref[...], wn[...], preferred_element_type=_f32)
         + bsb[...])
    r = jnp.maximum(_ln(o, g_ln[...], b_ln[...]), 0.0)
    score_out[...] = jnp.dot(r, wo[...], preferred_element_type=_f32) + bo[...]


@functools.lru_cache(maxsize=None)
def _tcc2():
    bs = pl.BlockSpec
    return pl.pallas_call(
        _tcc2_body,
        grid=(GRID,),
        in_specs=[
            bs((BN, H2), lambda i: (i, 0)),
            bs((BN, H2), lambda i: (i, 0)),
            bs((H2, H2), lambda i: (0, 0)),
            bs((H2, H2), lambda i: (0, 0)),
            bs((1, H2), lambda i: (0, 0)),
            bs((1, H2), lambda i: (0, 0)),
            bs((1, H2), lambda i: (0, 0)),
            bs((H2, 8), lambda i: (0, 0)),
            bs((1, 8), lambda i: (0, 0)),
        ],
        out_specs=[bs((BN, 8), lambda i: (i, 0))],
        out_shape=[jax.ShapeDtypeStruct((N, 8), _f32)],
        compiler_params=pltpu.CompilerParams(
            dimension_semantics=("arbitrary",)),
    )


def kernel(h, params, edge_index):
    p = params
    src = edge_index[0]
    dst = edge_index[1]

    npad = E_PAD - E
    dead = (jnp.arange(npad, dtype=_i32) % 96) + DEAD
    src_k1 = jnp.concatenate([src, dead]).reshape(ROWS, 128)
    dst_k1 = jnp.concatenate([dst, dead]).reshape(ROWS, 128)
    src_k3 = jnp.concatenate([src, jnp.zeros((npad,), _i32)]).reshape(ROWS, 128)

    deg_out, deg_in = _k1()(src_k1, dst_k1)
    hists = _k2a()(dst_k1)
    perm_src, perm_dst, offsets = _k2b()(dst_k1, src_k3, hists)

    r2 = lambda a: a.reshape(1, -1)
    x, xs, pool = _tca()(
        h, deg_out.reshape(N, 1),
        p['W_in'], r2(p['b_in']), p['W_t1'], r2(p['b_t1']),
        p['W_t2'], r2(p['b_t2']), p['W_gate'], r2(p['b_gate']))

    agg = _seg_kernel(H, False)(xs, perm_src, perm_dst, offsets)[:N]

    l1, l2 = p['layers'][0], p['layers'][1]
    z, hp1 = _tcb()(
        agg, deg_in.reshape(N, 1), x, pool,
        p['W_gcn'], r2(p['b_gcn']), r2(p['ln_gcn_g']), r2(p['ln_gcn_b']),
        l1['Wp'][:H], l1['Wp'][H:], r2(l1['bp']))

    n1 = _seg_kernel(H2, True)(hp1, perm_src, perm_dst, offsets)[:N]
    hh1, hp2 = _tcc1()(
        z, n1, l1['Ws'], l1['Wn'], r2(l1['bs']),
        r2(l1['ln_g']), r2(l1['ln_b']), l2['Wp'], r2(l2['bp']))

    n2 = _seg_kernel(H2, True)(hp2, perm_src, perm_dst, offsets)[:N]
    wo = jnp.pad(p['W_out'], ((0, 0), (0, 8 - OUT)))
    bo = jnp.pad(p['b_out'], (0, 8 - OUT)).reshape(1, 8)
    (score8,) = _tcc2()(
        hh1, n2, l2['Ws'], l2['Wn'], r2(l2['bs']),
        r2(l2['ln_g']), r2(l2['ln_b']), wo, bo)

    return score8[:, :OUT], z


# speedup vs baseline: 2.6105x; 1.3190x over previous
import functools

import jax
import jax.numpy as jnp
from jax import lax
from jax.experimental import pallas as pl
from jax.experimental.pallas import tpu as pltpu
from jax.experimental.pallas import tpu_sc as plsc

N = 100000
E = 1600000
DIN = 17
H = 64
H2 = 128
OUT = 2
NEG = 0.05
EPS = 1e-5

NC = 2
NS = 16
NW = NC * NS
L = 16

SHIFT = 9
CH = 1 << SHIFT
C = (N + CH - 1) // CH
NPAD = C * CH
CB = 256
DEAD = NPAD

E_PAD = 1605632
ROWS = E_PAD // 128
ROWS_T16 = ROWS // 16
ROWS_T32 = ROWS // 32
PERM_PAD = E_PAD + 256
ACC1 = 102400
W_E = 192

BN = 2000
GRID = N // BN

_i32 = jnp.int32
_f32 = jnp.float32


def _mesh():
    return plsc.VectorSubcoreMesh(core_axis_name="c", subcore_axis_name="s")


def _wid():
    return lax.axis_index("s") * NC + lax.axis_index("c")


def _iota():
    return lax.iota(_i32, L)


def _k1_body(src2d, dst2d, deg_out, deg_in, acc, idxbuf, onesbuf, zbuf):
    cid = lax.axis_index("c")
    sid = lax.axis_index("s")

    def zb(j, _):
        zbuf[pl.ds(j * L, L)] = jnp.zeros((L,), _f32)
        return 0
    lax.fori_loop(0, 6400 // L, zb, 0)
    pltpu.sync_copy(zbuf, acc.at[pl.ds(sid * 6400, 6400)])

    for j in range(128 // L):
        onesbuf[0, pl.ds(j * L, L)] = jnp.ones((L,), _f32)
    plsc.subcore_barrier()

    def accum(ref2d):
        base_row = sid * ROWS_T16

        def win_body(w, _):
            pltpu.sync_copy(ref2d.at[pl.ds(base_row + w * 16, 16)], idxbuf)
            for k in range(16):
                pltpu.sync_copy(onesbuf.at[0], acc.at[idxbuf.at[k]], add=True)
            return 0
        lax.fori_loop(0, ROWS_T16 // 16, win_body, 0)
        rem = ROWS_T16 % 16
        if rem:
            pltpu.sync_copy(ref2d.at[pl.ds(base_row + ROWS_T16 - rem, rem)],
                            idxbuf.at[pl.ds(0, rem)])
            for k in range(rem):
                pltpu.sync_copy(onesbuf.at[0], acc.at[idxbuf.at[k]], add=True)

    @pl.when(cid == 0)
    def _():
        accum(src2d)

    @pl.when(cid == 1)
    def _():
        accum(dst2d)

    plsc.subcore_barrier()

    def copy_out(dst_ref):
        n_per = 6400
        @pl.when(sid < NS - 1)
        def _():
            pltpu.sync_copy(acc.at[pl.ds(sid * n_per, n_per)], zbuf)
            pltpu.sync_copy(zbuf, dst_ref.at[pl.ds(sid * n_per, n_per)])
        last = N - (NS - 1) * n_per
        @pl.when(sid == NS - 1)
        def _():
            pltpu.sync_copy(acc.at[pl.ds((NS - 1) * n_per, last)],
                            zbuf.at[pl.ds(0, last)])
            pltpu.sync_copy(zbuf.at[pl.ds(0, last)],
                            dst_ref.at[pl.ds((NS - 1) * n_per, last)])

    @pl.when(cid == 0)
    def _():
        copy_out(deg_out)

    @pl.when(cid == 1)
    def _():
        copy_out(deg_in)


@functools.lru_cache(maxsize=None)
def _k1():
    return functools.partial(
        pl.kernel,
        mesh=_mesh(),
        compiler_params=pltpu.CompilerParams(needs_layout_passes=False),
        out_type=(jax.ShapeDtypeStruct((N,), _f32),
                  jax.ShapeDtypeStruct((N,), _f32)),
        scratch_types=[
            pltpu.VMEM_SHARED((ACC1,), _f32),
            pltpu.VMEM((16, 128), _i32),
            pltpu.VMEM((1, 128), _f32),
            pltpu.VMEM((6400,), _f32),
        ],
    )(_k1_body)


def _k2a_body(dst2d, hists, win, hist2, histv):
    wid = _wid()
    iota = _iota()
    lanebase = iota * CB
    ones = jnp.ones((L,), _i32)

    def zb(j, _):
        hist2[pl.ds(j * L, L)] = jnp.zeros((L,), _i32)
        return 0
    lax.fori_loop(0, (L * CB) // L, zb, 0)

    base_row = wid * ROWS_T32

    def hrow(k):
        for v in range(8):
            d16 = win[k, pl.ds(v * L, L)]
            b16 = lax.shift_right_logical(d16, SHIFT)
            plsc.addupdate_scatter(hist2, [lanebase + b16], ones)

    def win_body(w, _):
        pltpu.sync_copy(dst2d.at[pl.ds(base_row + w * 16, 16)], win)

        def row_body(k, _):
            hrow(k)
            return 0
        lax.fori_loop(0, 16, row_body, 0)
        return 0
    lax.fori_loop(0, ROWS_T32 // 16, win_body, 0)
    rem = ROWS_T32 % 16
    if rem:
        pltpu.sync_copy(dst2d.at[pl.ds(base_row + ROWS_T32 - rem, rem)],
                        win.at[pl.ds(0, rem)])
        def row_body_r(k, _):
            hrow(k)
            return 0
        lax.fori_loop(0, rem, row_body_r, 0)

    def red(g, _):
        s = hist2[pl.ds(g * L, L)]
        for r in range(1, 16):
            s = s + hist2[pl.ds(r * CB + g * L, L)]
        histv[pl.ds(g * L, L)] = s
        return 0
    lax.fori_loop(0, CB // L, red, 0)
    pltpu.sync_copy(histv, hists.at[pl.ds(wid * CB, CB)])


@functools.lru_cache(maxsize=None)
def _k2a():
    return functools.partial(
        pl.kernel,
        mesh=_mesh(),
        compiler_params=pltpu.CompilerParams(needs_layout_passes=False),
        out_type=jax.ShapeDtypeStruct((NW * CB,), _i32),
        scratch_types=[
            pltpu.VMEM((16, 128), _i32),
            pltpu.VMEM((L * CB,), _i32),
            pltpu.VMEM((CB,), _i32),
        ],
    )(_k2a_body)


def _k2b_body(dst2d, src2d, hists, perm_src, perm_dst, offsets,
              histsv, offs_all, cursors, dwin, swin,
              kbt, dt, st, posb, sb, db, offv, ztail, sem):
    wid = _wid()
    iota = _iota()

    pltpu.sync_copy(hists, histsv)

    def scan_body(j, carry):
        idx16 = j * L + iota
        b16 = lax.shift_right_logical(idx16, 5)
        t16 = idx16 & 31
        cnt = plsc.load_gather(histsv, [t16 * CB + b16])
        incl = plsc.cumsum(cnt)
        offs_all[pl.ds(j * L, L)] = incl - cnt + carry
        return carry + jnp.sum(cnt)
    lax.fori_loop(0, (CB * NW) // L, scan_body, jnp.int32(0))

    def cur_body(g, _):
        b16 = g * L + iota
        cursors[pl.ds(g * L, L)] = plsc.load_gather(offs_all, [b16 * NW + wid])
        return 0
    lax.fori_loop(0, CB // L, cur_body, 0)

    @pl.when(wid == 0)
    def _():
        def off_body(g, _):
            b16 = g * L + iota
            offv[pl.ds(g * L, L)] = plsc.load_gather(offs_all, [b16 * NW])
            return 0
        lax.fori_loop(0, CB // L, off_body, 0)
        pltpu.sync_copy(offv, offsets)

        def zt(j, _):
            ztail[pl.ds(j * L, L)] = jnp.zeros((L,), _i32)
            return 0
        lax.fori_loop(0, 256 // L, zt, 0)
        pltpu.sync_copy(ztail, perm_src.at[pl.ds(E_PAD, 256)])
        pltpu.sync_copy(ztail, perm_dst.at[pl.ds(E_PAD, 256)])

    base_row = wid * ROWS_T32

    def do_row(k):
        for v in range(8):
            d16 = dwin[k, pl.ds(v * L, L)]
            s16 = swin[k, pl.ds(v * L, L)]
            b16 = lax.shift_right_logical(d16, SHIFT)
            kb, vl = plsc.sort_key_val(b16, iota)
            kbt[...] = kb
            prev = plsc.load_gather(kbt, [jnp.maximum(iota - 1, 0)])
            nxt = plsc.load_gather(kbt, [jnp.minimum(iota + 1, L - 1)])
            isstart = (iota == 0) | (kb != prev)
            islast = (iota == L - 1) | (kb != nxt)
            runstart = plsc.cummax(jnp.where(isstart, iota, 0))
            rank = iota - runstart
            base = plsc.load_gather(cursors, [kb])
            pos = base + rank
            plsc.store_scatter(cursors, [kb], pos + 1, mask=islast)
            dt[...] = d16
            st[...] = s16
            dperm = plsc.load_gather(dt, [vl])
            sperm = plsc.load_gather(st, [vl])
            posb[k, pl.ds(v * L, L)] = pos
            sb[k, pl.ds(v * L, L)] = sperm
            db[k, pl.ds(v * L, L)] = dperm

    def flush(nrows):
        descs = []
        for k in range(nrows):
            descs.append(pltpu.async_copy(sb.at[k], perm_src.at[posb.at[k]],
                                          sem))
            descs.append(pltpu.async_copy(db.at[k], perm_dst.at[posb.at[k]],
                                          sem))
        for d in descs:
            d.wait()

    def win_body(w, _):
        pltpu.sync_copy(dst2d.at[pl.ds(base_row + w * 16, 16)], dwin)
        pltpu.sync_copy(src2d.at[pl.ds(base_row + w * 16, 16)], swin)

        def row_body(k, _):
            do_row(k)
            return 0
        lax.fori_loop(0, 16, row_body, 0)
        flush(16)
        return 0
    lax.fori_loop(0, ROWS_T32 // 16, win_body, 0)

    rem = ROWS_T32 % 16
    if rem:
        pltpu.sync_copy(dst2d.at[pl.ds(base_row + ROWS_T32 - rem, rem)],
                        dwin.at[pl.ds(0, rem)])
        pltpu.sync_copy(src2d.at[pl.ds(base_row + ROWS_T32 - rem, rem)],
                        swin.at[pl.ds(0, rem)])

        def row_body_r(k, _):
            do_row(k)
            return 0
        lax.fori_loop(0, rem, row_body_r, 0)
        flush(rem)


@functools.lru_cache(maxsize=None)
def _k2b():
    return functools.partial(
        pl.kernel,
        mesh=_mesh(),
        compiler_params=pltpu.CompilerParams(needs_layout_passes=False),
        out_type=(jax.ShapeDtypeStruct((PERM_PAD,), _i32),
                  jax.ShapeDtypeStruct((PERM_PAD,), _i32),
                  jax.ShapeDtypeStruct((CB,), _i32)),
        scratch_types=[
            pltpu.VMEM((NW * CB,), _i32),
            pltpu.VMEM((CB * NW,), _i32),
            pltpu.VMEM((CB,), _i32),
            pltpu.VMEM((16, 128), _i32),
            pltpu.VMEM((16, 128), _i32),
            pltpu.VMEM((L,), _i32),
            pltpu.VMEM((L,), _i32),
            pltpu.VMEM((L,), _i32),
            pltpu.VMEM((16, 128), _i32),
            pltpu.VMEM((16, 128), _i32),
            pltpu.VMEM((16, 128), _i32),
            pltpu.VMEM((CB,), _i32),
            pltpu.VMEM((256,), _i32),
            pltpu.SemaphoreType.DMA,
        ],
    )(_k2b_body)


def _make_seg_body(width, is_max):
    nj = width // L

    def body(feat, perm_src, perm_dst, offsets, out, acc,
             gbuf0, gbuf1, swin0, swin1, dwin0, dwin1, offv, sem0, sem1):
        wid = _wid()
        iota = _iota()
        pltpu.sync_copy(offsets, offv)
        bufs = ((gbuf0, swin0, dwin0, sem0), (gbuf1, swin1, dwin1, sem1))

        def run_chunk(c):
            def zb(i, _):
                for j in range(nj):
                    acc[i, pl.ds(j * L, L)] = jnp.zeros((L,), _f32)
                return 0
            lax.fori_loop(0, CH, zb, 0)

            off2 = offv[pl.ds(c, L)]
            start0 = off2[0]
            end0 = off2[1]
            astart = start0 & jnp.int32(-8)
            nwin = (end0 - astart + (W_E - 1)) // W_E
            cbase = c * CH

            def stage_fire(w, gbuf, swin, dwin, sem):
                ws = pl.multiple_of(astart + w * W_E, 8)
                pltpu.sync_copy(perm_src.at[pl.ds(ws, W_E)], swin)
                pltpu.sync_copy(perm_dst.at[pl.ds(ws, W_E)], dwin)
                pltpu.async_copy(feat.at[swin], gbuf, sem)

            for b in range(2):
                @pl.when(b < nwin)
                def _():
                    stage_fire(jnp.int32(b), *bufs[b])

            def rmw(w, gbuf, dwin):
                ws = astart + w * W_E
                lo = start0 - ws
                hi = end0 - ws

                def blk_body(blk, _):
                    eidx = blk * L + iota
                    inr = (eidx >= lo) & (eidx < hi)
                    d16 = dwin[pl.ds(blk * L, L)]
                    dl16 = jnp.where(inr, d16 - cbase, CH)
                    for u in range(L):
                        dloc = dl16[u]
                        for j in range(nj):
                            cur = acc[dloc, pl.ds(j * L, L)]
                            val = gbuf[blk * L + u, pl.ds(j * L, L)]
                            if is_max:
                                acc[dloc, pl.ds(j * L, L)] = \
                                    jnp.maximum(cur, val)
                            else:
                                acc[dloc, pl.ds(j * L, L)] = cur + val
                    return 0
                lax.fori_loop(0, W_E // L, blk_body, 0)

            def pair_body(g2, _):
                for b in range(2):
                    w = g2 * 2 + b
                    gbuf, swin, dwin, sem = bufs[b]

                    @pl.when(w < nwin)
                    def _():
                        pltpu.make_async_copy(feat.at[swin], gbuf, sem).wait()
                        rmw(w, gbuf, dwin)

                    @pl.when(w + 2 < nwin)
                    def _():
                        stage_fire(w + 2, gbuf, swin, dwin, sem)
                return 0
            lax.fori_loop(0, (nwin + 1) // 2, pair_body, 0)

            pltpu.sync_copy(acc.at[pl.ds(0, CH)], out.at[pl.ds(cbase, CH)])

        for t in range(C // NW):
            run_chunk(wid + t * NW)
        remc = C % NW
        if remc:
            @pl.when(wid < remc)
            def _():
                run_chunk(wid + (C // NW) * NW)

    return body


@functools.lru_cache(maxsize=None)
def _seg_kernel(width, is_max):
    return functools.partial(
        pl.kernel,
        mesh=_mesh(),
        compiler_params=pltpu.CompilerParams(needs_layout_passes=False),
        out_type=jax.ShapeDtypeStruct((NPAD, width), _f32),
        scratch_types=[
            pltpu.VMEM((CH + 8, width), _f32),
            pltpu.VMEM((W_E, H2), _f32),
            pltpu.VMEM((W_E, H2), _f32),
            pltpu.VMEM((W_E,), _i32),
            pltpu.VMEM((W_E,), _i32),
            pltpu.VMEM((W_E,), _i32),
            pltpu.VMEM((W_E,), _i32),
            pltpu.VMEM((CB,), _i32),
            pltpu.SemaphoreType.DMA,
            pltpu.SemaphoreType.DMA,
        ],
    )(_make_seg_body(width, is_max))


def _ln(x, g, b):
    m = jnp.mean(x, axis=-1, keepdims=True)
    v = jnp.mean((x - m) * (x - m), axis=-1, keepdims=True)
    return (x - m) * lax.rsqrt(v + EPS) * g + b


def _tca_body(h_ref, deg_ref, w_in, b_in, w_t1, b_t1, w_t2, b_t2, w_g, b_g,
              x_out, xs_out, pool_out, m_ref, s_ref, p_ref):
    i = pl.program_id(0)
    hb = h_ref[...]
    x = jnp.dot(hb, w_in[...], preferred_element_type=_f32) + b_in[...]
    x = jnp.dot(x, w_t1[...], preferred_element_type=_f32) + b_t1[...]
    x = jnp.where(x >= 0, x, NEG * x)
    x = jnp.dot(x, w_t2[...], preferred_element_type=_f32) + b_t2[...]
    x_out[...] = x
    deg = deg_ref[...]
    xs = x * lax.rsqrt(jnp.maximum(deg, 1.0))
    xs_out[...] = jnp.concatenate([xs, jnp.zeros_like(xs)], axis=1)

    g = jnp.dot(x, w_g[...], preferred_element_type=_f32) + b_g[...]

    @pl.when(i == 0)
    def _():
        m_ref[0] = -jnp.inf
        s_ref[0] = 0.0
        p_ref[...] = jnp.zeros_like(p_ref)

    bm = jnp.max(g)
    mo = m_ref[0]
    mn = jnp.maximum(mo, bm)
    corr = jnp.exp(mo - mn)
    e = jnp.exp(g - mn)
    s_new = s_ref[0] * corr + jnp.sum(e)
    s_ref[0] = s_new
    p_new = p_ref[...] * corr + jnp.sum(e * x, axis=0, keepdims=True)
    p_ref[...] = p_new
    pool_out[...] = p_new / s_new


@functools.lru_cache(maxsize=None)
def _tca():
    bs = pl.BlockSpec
    return pl.pallas_call(
        _tca_body,
        grid=(GRID,),
        in_specs=[
            bs((BN, DIN), lambda i: (i, 0)),
            bs((BN, 1), lambda i: (i, 0)),
            bs((DIN, H), lambda i: (0, 0)),
            bs((1, H), lambda i: (0, 0)),
            bs((H, H), lambda i: (0, 0)),
            bs((1, H), lambda i: (0, 0)),
            bs((H, H), lambda i: (0, 0)),
            bs((1, H), lambda i: (0, 0)),
            bs((H, 1), lambda i: (0, 0)),
            bs((1, 1), lambda i: (0, 0)),
        ],
        out_specs=[
            bs((BN, H), lambda i: (i, 0)),
            bs((BN, H2), lambda i: (i, 0)),
            bs((1, H), lambda i: (0, 0)),
        ],
        out_shape=[
            jax.ShapeDtypeStruct((N, H), _f32),
            jax.ShapeDtypeStruct((N, H2), _f32),
            jax.ShapeDtypeStruct((1, H), _f32),
        ],
        scratch_shapes=[
            pltpu.SMEM((1,), _f32),
            pltpu.SMEM((1,), _f32),
            pltpu.VMEM((1, H), _f32),
        ],
        compiler_params=pltpu.CompilerParams(
            dimension_semantics=("arbitrary",)),
    )


def _tcb_body(agg_ref, deg_ref, x_ref, pool_ref, w_gcn, b_gcn, g_gcn, bn_gcn,
              wp1a, wp1b, bp1, z_out, hp1_out):
    a = agg_ref[...] * lax.rsqrt(jnp.maximum(deg_ref[...], 1.0))
    t = jnp.dot(a, w_gcn[...], preferred_element_type=_f32) + b_gcn[...]
    g1 = _ln(t, g_gcn[...], bn_gcn[...])
    xb = x_ref[...]
    loc = g1 - xb
    glo = pool_ref[...] - xb
    z_out[...] = jnp.concatenate([loc, glo], axis=1)
    hp = (jnp.dot(loc, wp1a[...], preferred_element_type=_f32)
          + jnp.dot(glo, wp1b[...], preferred_element_type=_f32) + bp1[...])
    hp1_out[...] = jnp.maximum(hp, 0.0)


@functools.lru_cache(maxsize=None)
def _tcb():
    bs = pl.BlockSpec
    return pl.pallas_call(
        _tcb_body,
        grid=(GRID,),
        in_specs=[
            bs((BN, H), lambda i: (i, 0)),
            bs((BN, 1), lambda i: (i, 0)),
            bs((BN, H), lambda i: (i, 0)),
            bs((1, H), lambda i: (0, 0)),
            bs((H, H), lambda i: (0, 0)),
            bs((1, H), lambda i: (0, 0)),
            bs((1, H), lambda i: (0, 0)),
            bs((1, H), lambda i: (0, 0)),
            bs((H, H2), lambda i: (0, 0)),
            bs((H, H2), lambda i: (0, 0)),
            bs((1, H2), lambda i: (0, 0)),
        ],
        out_specs=[
            bs((BN, H2), lambda i: (i, 0)),
            bs((BN, H2), lambda i: (i, 0)),
        ],
        out_shape=[
            jax.ShapeDtypeStruct((N, H2), _f32),
            jax.ShapeDtypeStruct((N, H2), _f32),
        ],
        compiler_params=pltpu.CompilerParams(
            dimension_semantics=("arbitrary",)),
    )


def _tcc1_body(hh_ref, n_ref, ws, wn, bsb, g_ln, b_ln, wp2, bp2,
               hh1_out, hp2_out):
    o = (jnp.dot(hh_ref[...], ws[...], preferred_element_type=_f32)
         + jnp.dot(n_ref[...], wn[...], preferred_element_type=_f32)
         + bsb[...])
    r = jnp.maximum(_ln(o, g_ln[...], b_ln[...]), 0.0)
    hh1_out[...] = r
    hp = jnp.dot(r, wp2[...], preferred_element_type=_f32) + bp2[...]
    hp2_out[...] = jnp.maximum(hp, 0.0)


@functools.lru_cache(maxsize=None)
def _tcc1():
    bs = pl.BlockSpec
    return pl.pallas_call(
        _tcc1_body,
        grid=(GRID,),
        in_specs=[
            bs((BN, H2), lambda i: (i, 0)),
            bs((BN, H2), lambda i: (i, 0)),
            bs((H2, H2), lambda i: (0, 0)),
            bs((H2, H2), lambda i: (0, 0)),
            bs((1, H2), lambda i: (0, 0)),
            bs((1, H2), lambda i: (0, 0)),
            bs((1, H2), lambda i: (0, 0)),
            bs((H2, H2), lambda i: (0, 0)),
            bs((1, H2), lambda i: (0, 0)),
        ],
        out_specs=[
            bs((BN, H2), lambda i: (i, 0)),
            bs((BN, H2), lambda i: (i, 0)),
        ],
        out_shape=[
            jax.ShapeDtypeStruct((N, H2), _f32),
            jax.ShapeDtypeStruct((N, H2), _f32),
        ],
        compiler_params=pltpu.CompilerParams(
            dimension_semantics=("arbitrary",)),
    )


def _tcc2_body(hh_ref, n_ref, ws, wn, bsb, g_ln, b_ln, wo, bo, score_out):
    o = (jnp.dot(hh_ref[...], ws[...], preferred_element_type=_f32)
         + jnp.dot(n_ref[...], wn[...], preferred_element_type=_f32)
         + bsb[...])
    r = jnp.maximum(_ln(o, g_ln[...], b_ln[...]), 0.0)
    score_out[...] = jnp.dot(r, wo[...], preferred_element_type=_f32) + bo[...]


@functools.lru_cache(maxsize=None)
def _tcc2():
    bs = pl.BlockSpec
    return pl.pallas_call(
        _tcc2_body,
        grid=(GRID,),
        in_specs=[
            bs((BN, H2), lambda i: (i, 0)),
            bs((BN, H2), lambda i: (i, 0)),
            bs((H2, H2), lambda i: (0, 0)),
            bs((H2, H2), lambda i: (0, 0)),
            bs((1, H2), lambda i: (0, 0)),
            bs((1, H2), lambda i: (0, 0)),
            bs((1, H2), lambda i: (0, 0)),
            bs((H2, 8), lambda i: (0, 0)),
            bs((1, 8), lambda i: (0, 0)),
        ],
        out_specs=[bs((BN, 8), lambda i: (i, 0))],
        out_shape=[jax.ShapeDtypeStruct((N, 8), _f32)],
        compiler_params=pltpu.CompilerParams(
            dimension_semantics=("arbitrary",)),
    )


def kernel(h, params, edge_index):
    p = params
    src = edge_index[0]
    dst = edge_index[1]

    npad = E_PAD - E
    dead = (jnp.arange(npad, dtype=_i32) % 96) + DEAD
    src_k1 = jnp.concatenate([src, dead]).reshape(ROWS, 128)
    dst_k1 = jnp.concatenate([dst, dead]).reshape(ROWS, 128)
    src_k3 = jnp.concatenate([src, jnp.zeros((npad,), _i32)]).reshape(ROWS, 128)

    deg_out, deg_in = _k1()(src_k1, dst_k1)
    hists = _k2a()(dst_k1)
    perm_src, perm_dst, offsets = _k2b()(dst_k1, src_k3, hists)

    r2 = lambda a: a.reshape(1, -1)
    x, xs, pool = _tca()(
        h, deg_out.reshape(N, 1),
        p['W_in'], r2(p['b_in']), p['W_t1'], r2(p['b_t1']),
        p['W_t2'], r2(p['b_t2']), p['W_gate'], r2(p['b_gate']))

    agg = _seg_kernel(H, False)(xs, perm_src, perm_dst, offsets)[:N]

    l1, l2 = p['layers'][0], p['layers'][1]
    z, hp1 = _tcb()(
        agg, deg_in.reshape(N, 1), x, pool,
        p['W_gcn'], r2(p['b_gcn']), r2(p['ln_gcn_g']), r2(p['ln_gcn_b']),
        l1['Wp'][:H], l1['Wp'][H:], r2(l1['bp']))

    n1 = _seg_kernel(H2, True)(hp1, perm_src, perm_dst, offsets)[:N]
    hh1, hp2 = _tcc1()(
        z, n1, l1['Ws'], l1['Wn'], r2(l1['bs']),
        r2(l1['ln_g']), r2(l1['ln_b']), l2['Wp'], r2(l2['bp']))

    n2 = _seg_kernel(H2, True)(hp2, perm_src, perm_dst, offsets)[:N]
    wo = jnp.pad(p['W_out'], ((0, 0), (0, 8 - OUT)))
    bo = jnp.pad(p['b_out'], (0, 8 - OUT)).reshape(1, 8)
    (score8,) = _tcc2()(
        hh1, n2, l2['Ws'], l2['Wn'], r2(l2['bs']),
        r2(l2['ln_g']), r2(l2['ln_b']), wo, bo)

    return score8[:, :OUT], z


# speedup vs baseline: 5.1493x; 1.9726x over previous
import functools

import jax
import jax.numpy as jnp
from jax import lax
from jax.experimental import pallas as pl
from jax.experimental.pallas import tpu as pltpu
from jax.experimental.pallas import tpu_sc as plsc

N = 100000
E = 1600000
DIN = 17
H = 64
H2 = 128
OUT = 2
NEG = 0.05
EPS = 1e-5

NC = 2
NS = 16
NW = NC * NS
L = 16

SHIFT = 9
CH = 1 << SHIFT
C = (N + CH - 1) // CH
NPAD = C * CH
CB = 256
DEAD = NPAD

E_PAD = 1605632
ROWS = E_PAD // 128
ROWS_T16 = ROWS // 16
ROWS_T32 = ROWS // 32
PERM_PAD = E_PAD + 256
ACC1 = 102400
W_E = 192

BN = 2000
GRID = N // BN

_i32 = jnp.int32
_f32 = jnp.float32


def _mesh():
    return plsc.VectorSubcoreMesh(core_axis_name="c", subcore_axis_name="s")


def _wid():
    return lax.axis_index("s") * NC + lax.axis_index("c")


def _iota():
    return lax.iota(_i32, L)


def _k1_body(src2d, dst2d, deg_out, deg_in, acc, idxbuf, onesbuf, zbuf):
    cid = lax.axis_index("c")
    sid = lax.axis_index("s")

    def zb(j, _):
        zbuf[pl.ds(j * L, L)] = jnp.zeros((L,), _f32)
        return 0
    lax.fori_loop(0, 6400 // L, zb, 0)
    pltpu.sync_copy(zbuf, acc.at[pl.ds(sid * 6400, 6400)])

    for j in range(128 // L):
        onesbuf[0, pl.ds(j * L, L)] = jnp.ones((L,), _f32)
    plsc.subcore_barrier()

    def accum(ref2d):
        base_row = sid * ROWS_T16

        def win_body(w, _):
            pltpu.sync_copy(ref2d.at[pl.ds(base_row + w * 16, 16)], idxbuf)
            for k in range(16):
                pltpu.sync_copy(onesbuf.at[0], acc.at[idxbuf.at[k]], add=True)
            return 0
        lax.fori_loop(0, ROWS_T16 // 16, win_body, 0)
        rem = ROWS_T16 % 16
        if rem:
            pltpu.sync_copy(ref2d.at[pl.ds(base_row + ROWS_T16 - rem, rem)],
                            idxbuf.at[pl.ds(0, rem)])
            for k in range(rem):
                pltpu.sync_copy(onesbuf.at[0], acc.at[idxbuf.at[k]], add=True)

    @pl.when(cid == 0)
    def _():
        accum(src2d)

    @pl.when(cid == 1)
    def _():
        accum(dst2d)

    plsc.subcore_barrier()

    def copy_out(dst_ref):
        n_per = 6400
        @pl.when(sid < NS - 1)
        def _():
            pltpu.sync_copy(acc.at[pl.ds(sid * n_per, n_per)], zbuf)
            pltpu.sync_copy(zbuf, dst_ref.at[pl.ds(sid * n_per, n_per)])
        last = N - (NS - 1) * n_per
        @pl.when(sid == NS - 1)
        def _():
            pltpu.sync_copy(acc.at[pl.ds((NS - 1) * n_per, last)],
                            zbuf.at[pl.ds(0, last)])
            pltpu.sync_copy(zbuf.at[pl.ds(0, last)],
                            dst_ref.at[pl.ds((NS - 1) * n_per, last)])

    @pl.when(cid == 0)
    def _():
        copy_out(deg_out)

    @pl.when(cid == 1)
    def _():
        copy_out(deg_in)


@functools.lru_cache(maxsize=None)
def _k1():
    return functools.partial(
        pl.kernel,
        mesh=_mesh(),
        compiler_params=pltpu.CompilerParams(needs_layout_passes=False),
        out_type=(jax.ShapeDtypeStruct((N,), _f32),
                  jax.ShapeDtypeStruct((N,), _f32)),
        scratch_types=[
            pltpu.VMEM_SHARED((ACC1,), _f32),
            pltpu.VMEM((16, 128), _i32),
            pltpu.VMEM((1, 128), _f32),
            pltpu.VMEM((6400,), _f32),
        ],
    )(_k1_body)


def _k2a_body(dst2d, hists, win, hist2, histv):
    wid = _wid()
    iota = _iota()
    lanebase = iota * CB
    ones = jnp.ones((L,), _i32)

    def zb(j, _):
        hist2[pl.ds(j * L, L)] = jnp.zeros((L,), _i32)
        return 0
    lax.fori_loop(0, (L * CB) // L, zb, 0)

    base_row = wid * ROWS_T32

    def hrow(k):
        for v in range(8):
            d16 = win[k, pl.ds(v * L, L)]
            b16 = lax.shift_right_logical(d16, SHIFT)
            plsc.addupdate_scatter(hist2, [lanebase + b16], ones)

    def win_body(w, _):
        pltpu.sync_copy(dst2d.at[pl.ds(base_row + w * 16, 16)], win)

        def row_body(k, _):
            hrow(k)
            return 0
        lax.fori_loop(0, 16, row_body, 0)
        return 0
    lax.fori_loop(0, ROWS_T32 // 16, win_body, 0)
    rem = ROWS_T32 % 16
    if rem:
        pltpu.sync_copy(dst2d.at[pl.ds(base_row + ROWS_T32 - rem, rem)],
                        win.at[pl.ds(0, rem)])
        def row_body_r(k, _):
            hrow(k)
            return 0
        lax.fori_loop(0, rem, row_body_r, 0)

    def red(g, _):
        s = hist2[pl.ds(g * L, L)]
        for r in range(1, 16):
            s = s + hist2[pl.ds(r * CB + g * L, L)]
        histv[pl.ds(g * L, L)] = s
        return 0
    lax.fori_loop(0, CB // L, red, 0)
    pltpu.sync_copy(histv, hists.at[pl.ds(wid * CB, CB)])


@functools.lru_cache(maxsize=None)
def _k2a():
    return functools.partial(
        pl.kernel,
        mesh=_mesh(),
        compiler_params=pltpu.CompilerParams(needs_layout_passes=False),
        out_type=jax.ShapeDtypeStruct((NW * CB,), _i32),
        scratch_types=[
            pltpu.VMEM((16, 128), _i32),
            pltpu.VMEM((L * CB,), _i32),
            pltpu.VMEM((CB,), _i32),
        ],
    )(_k2a_body)


def _k2b_body(dst2d, src2d, hists, perm_comb, offsets,
              histsv, offs_all, cursors, dwin, swin,
              kbt, ct, posb, cb, offv, ztail, sem):
    wid = _wid()
    iota = _iota()

    pltpu.sync_copy(hists, histsv)

    def scan_body(j, carry):
        idx16 = j * L + iota
        b16 = lax.shift_right_logical(idx16, 5)
        t16 = idx16 & 31
        cnt = plsc.load_gather(histsv, [t16 * CB + b16])
        incl = plsc.cumsum(cnt)
        offs_all[pl.ds(j * L, L)] = incl - cnt + carry
        return carry + jnp.sum(cnt)
    lax.fori_loop(0, (CB * NW) // L, scan_body, jnp.int32(0))

    def cur_body(g, _):
        b16 = g * L + iota
        cursors[pl.ds(g * L, L)] = plsc.load_gather(offs_all, [b16 * NW + wid])
        return 0
    lax.fori_loop(0, CB // L, cur_body, 0)

    @pl.when(wid == 0)
    def _():
        def off_body(g, _):
            b16 = g * L + iota
            offv[pl.ds(g * L, L)] = plsc.load_gather(offs_all, [b16 * NW])
            return 0
        lax.fori_loop(0, CB // L, off_body, 0)
        pltpu.sync_copy(offv, offsets)

        def zt(j, _):
            ztail[pl.ds(j * L, L)] = jnp.zeros((L,), _i32)
            return 0
        lax.fori_loop(0, 256 // L, zt, 0)
        pltpu.sync_copy(ztail, perm_comb.at[pl.ds(E_PAD, 256)])

    base_row = wid * ROWS_T32

    def do_row(k):
        for v in range(8):
            d16 = dwin[k, pl.ds(v * L, L)]
            s16 = swin[k, pl.ds(v * L, L)]
            b16 = lax.shift_right_logical(d16, SHIFT)
            comb = lax.shift_left(s16, SHIFT) | (d16 & (CH - 1))
            kb, vl = plsc.sort_key_val(b16, iota)
            kbt[...] = kb
            prev = plsc.load_gather(kbt, [jnp.maximum(iota - 1, 0)])
            nxt = plsc.load_gather(kbt, [jnp.minimum(iota + 1, L - 1)])
            isstart = (iota == 0) | (kb != prev)
            islast = (iota == L - 1) | (kb != nxt)
            runstart = plsc.cummax(jnp.where(isstart, iota, 0))
            rank = iota - runstart
            base = plsc.load_gather(cursors, [kb])
            pos = base + rank
            plsc.store_scatter(cursors, [kb], pos + 1, mask=islast)
            ct[...] = comb
            cperm = plsc.load_gather(ct, [vl])
            posb[k, pl.ds(v * L, L)] = pos
            cb[k, pl.ds(v * L, L)] = cperm

    def flush(nrows):
        descs = []
        for k in range(nrows):
            descs.append(pltpu.async_copy(cb.at[k], perm_comb.at[posb.at[k]],
                                          sem))
        for d in descs:
            d.wait()

    def win_body(w, _):
        pltpu.sync_copy(dst2d.at[pl.ds(base_row + w * 16, 16)], dwin)
        pltpu.sync_copy(src2d.at[pl.ds(base_row + w * 16, 16)], swin)

        def row_body(k, _):
            do_row(k)
            return 0
        lax.fori_loop(0, 16, row_body, 0)
        flush(16)
        return 0
    lax.fori_loop(0, ROWS_T32 // 16, win_body, 0)

    rem = ROWS_T32 % 16
    if rem:
        pltpu.sync_copy(dst2d.at[pl.ds(base_row + ROWS_T32 - rem, rem)],
                        dwin.at[pl.ds(0, rem)])
        pltpu.sync_copy(src2d.at[pl.ds(base_row + ROWS_T32 - rem, rem)],
                        swin.at[pl.ds(0, rem)])

        def row_body_r(k, _):
            do_row(k)
            return 0
        lax.fori_loop(0, rem, row_body_r, 0)
        flush(rem)


@functools.lru_cache(maxsize=None)
def _k2b():
    return functools.partial(
        pl.kernel,
        mesh=_mesh(),
        compiler_params=pltpu.CompilerParams(needs_layout_passes=False),
        out_type=(jax.ShapeDtypeStruct((PERM_PAD,), _i32),
                  jax.ShapeDtypeStruct((CB,), _i32)),
        scratch_types=[
            pltpu.VMEM((NW * CB,), _i32),
            pltpu.VMEM((CB * NW,), _i32),
            pltpu.VMEM((CB,), _i32),
            pltpu.VMEM((16, 128), _i32),
            pltpu.VMEM((16, 128), _i32),
            pltpu.VMEM((L,), _i32),
            pltpu.VMEM((L,), _i32),
            pltpu.VMEM((16, 128), _i32),
            pltpu.VMEM((16, 128), _i32),
            pltpu.VMEM((CB,), _i32),
            pltpu.VMEM((256,), _i32),
            pltpu.SemaphoreType.DMA,
        ],
    )(_k2b_body)


def _make_seg_body(width, is_max):
    nj = width // L

    def body(feat, perm_comb, offsets, out, acc,
             gbuf0, gbuf1, sidx0, sidx1, cwin0, cwin1, offv, sem0, sem1):
        wid = _wid()
        iota = _iota()
        pltpu.sync_copy(offsets, offv)
        bufs = ((gbuf0, sidx0, cwin0, sem0), (gbuf1, sidx1, cwin1, sem1))

        def run_chunk(c):
            def zb(i, _):
                for j in range(nj):
                    acc[i, pl.ds(j * L, L)] = jnp.zeros((L,), _f32)
                return 0
            lax.fori_loop(0, CH, zb, 0)

            off2 = offv[pl.ds(c, L)]
            start0 = off2[0]
            end0 = off2[1]
            astart = start0 & jnp.int32(-8)
            nwin = (end0 - astart + (W_E - 1)) // W_E
            cbase = c * CH

            def stage_fire(w, gbuf, sidx, cwin, sem):
                ws = pl.multiple_of(astart + w * W_E, 8)
                pltpu.sync_copy(perm_comb.at[pl.ds(ws, W_E)], cwin)

                def sx(b, _):
                    cv = cwin[pl.ds(b * L, L)]
                    sidx[pl.ds(b * L, L)] = lax.shift_right_logical(cv, SHIFT)
                    return 0
                lax.fori_loop(0, W_E // L, sx, 0)
                pltpu.async_copy(feat.at[sidx], gbuf, sem)

            for b in range(2):
                @pl.when(b < nwin)
                def _():
                    stage_fire(jnp.int32(b), *bufs[b])

            def rmw(w, gbuf, cwin):
                ws = astart + w * W_E
                lo = start0 - ws
                hi = end0 - ws

                def blk_body(blk, _):
                    eidx = blk * L + iota
                    inr = (eidx >= lo) & (eidx < hi)
                    d16 = cwin[pl.ds(blk * L, L)] & (CH - 1)
                    dl16 = jnp.where(inr, d16, CH)
                    for u2 in range(L // 2):
                        d_a = dl16[2 * u2]
                        d_b = dl16[2 * u2 + 1]
                        curs_a = [acc[d_a, pl.ds(j * L, L)] for j in range(nj)]
                        vals_a = [gbuf[blk * L + 2 * u2, pl.ds(j * L, L)]
                                  for j in range(nj)]
                        for j in range(nj):
                            if is_max:
                                acc[d_a, pl.ds(j * L, L)] = \
                                    jnp.maximum(curs_a[j], vals_a[j])
                            else:
                                acc[d_a, pl.ds(j * L, L)] = \
                                    curs_a[j] + vals_a[j]
                        curs_b = [acc[d_b, pl.ds(j * L, L)] for j in range(nj)]
                        vals_b = [gbuf[blk * L + 2 * u2 + 1, pl.ds(j * L, L)]
                                  for j in range(nj)]
                        for j in range(nj):
                            if is_max:
                                acc[d_b, pl.ds(j * L, L)] = \
                                    jnp.maximum(curs_b[j], vals_b[j])
                            else:
                                acc[d_b, pl.ds(j * L, L)] = \
                                    curs_b[j] + vals_b[j]
                    return 0
                lax.fori_loop(0, W_E // L, blk_body, 0)

            def pair_body(g2, _):
                for b in range(2):
                    w = g2 * 2 + b
                    gbuf, sidx, cwin, sem = bufs[b]

                    @pl.when(w < nwin)
                    def _():
                        pltpu.make_async_copy(feat.at[sidx], gbuf, sem).wait()
                        rmw(w, gbuf, cwin)

                    @pl.when(w + 2 < nwin)
                    def _():
                        stage_fire(w + 2, gbuf, sidx, cwin, sem)
                return 0
            lax.fori_loop(0, (nwin + 1) // 2, pair_body, 0)

            pltpu.sync_copy(acc.at[pl.ds(0, CH)], out.at[pl.ds(cbase, CH)])

        for t in range(C // NW):
            run_chunk(wid + t * NW)
        remc = C % NW
        if remc:
            @pl.when(wid < remc)
            def _():
                run_chunk(wid + (C // NW) * NW)

    return body


@functools.lru_cache(maxsize=None)
def _seg_kernel(width, is_max):
    return functools.partial(
        pl.kernel,
        mesh=_mesh(),
        compiler_params=pltpu.CompilerParams(needs_layout_passes=False),
        out_type=jax.ShapeDtypeStruct((NPAD, width), _f32),
        scratch_types=[
            pltpu.VMEM((CH + 8, width), _f32),
            pltpu.VMEM((W_E, H2), _f32),
            pltpu.VMEM((W_E, H2), _f32),
            pltpu.VMEM((W_E,), _i32),
            pltpu.VMEM((W_E,), _i32),
            pltpu.VMEM((W_E,), _i32),
            pltpu.VMEM((W_E,), _i32),
            pltpu.VMEM((CB,), _i32),
            pltpu.SemaphoreType.DMA,
            pltpu.SemaphoreType.DMA,
        ],
    )(_make_seg_body(width, is_max))


def _ln(x, g, b):
    m = jnp.mean(x, axis=-1, keepdims=True)
    v = jnp.mean((x - m) * (x - m), axis=-1, keepdims=True)
    return (x - m) * lax.rsqrt(v + EPS) * g + b


def _tca_body(h_ref, deg_ref, w_in, b_in, w_t1, b_t1, w_t2, b_t2, w_g, b_g,
              x_out, xs_out, pool_out, m_ref, s_ref, p_ref):
    i = pl.program_id(0)
    hb = h_ref[...]
    x = jnp.dot(hb, w_in[...], preferred_element_type=_f32) + b_in[...]
    x = jnp.dot(x, w_t1[...], preferred_element_type=_f32) + b_t1[...]
    x = jnp.where(x >= 0, x, NEG * x)
    x = jnp.dot(x, w_t2[...], preferred_element_type=_f32) + b_t2[...]
    x_out[...] = x
    deg = deg_ref[...]
    xs = x * lax.rsqrt(jnp.maximum(deg, 1.0))
    xs_out[...] = jnp.concatenate([xs, jnp.zeros_like(xs)], axis=1)

    g = jnp.dot(x, w_g[...], preferred_element_type=_f32) + b_g[...]

    @pl.when(i == 0)
    def _():
        m_ref[0] = -jnp.inf
        s_ref[0] = 0.0
        p_ref[...] = jnp.zeros_like(p_ref)

    bm = jnp.max(g)
    mo = m_ref[0]
    mn = jnp.maximum(mo, bm)
    corr = jnp.exp(mo - mn)
    e = jnp.exp(g - mn)
    s_new = s_ref[0] * corr + jnp.sum(e)
    s_ref[0] = s_new
    p_new = p_ref[...] * corr + jnp.sum(e * x, axis=0, keepdims=True)
    p_ref[...] = p_new
    pool_out[...] = p_new / s_new


@functools.lru_cache(maxsize=None)
def _tca():
    bs = pl.BlockSpec
    return pl.pallas_call(
        _tca_body,
        grid=(GRID,),
        in_specs=[
            bs((BN, DIN), lambda i: (i, 0)),
            bs((BN, 1), lambda i: (i, 0)),
            bs((DIN, H), lambda i: (0, 0)),
            bs((1, H), lambda i: (0, 0)),
            bs((H, H), lambda i: (0, 0)),
            bs((1, H), lambda i: (0, 0)),
            bs((H, H), lambda i: (0, 0)),
            bs((1, H), lambda i: (0, 0)),
            bs((H, 1), lambda i: (0, 0)),
            bs((1, 1), lambda i: (0, 0)),
        ],
        out_specs=[
            bs((BN, H), lambda i: (i, 0)),
            bs((BN, H2), lambda i: (i, 0)),
            bs((1, H), lambda i: (0, 0)),
        ],
        out_shape=[
            jax.ShapeDtypeStruct((N, H), _f32),
            jax.ShapeDtypeStruct((N, H2), _f32),
            jax.ShapeDtypeStruct((1, H), _f32),
        ],
        scratch_shapes=[
            pltpu.SMEM((1,), _f32),
            pltpu.SMEM((1,), _f32),
            pltpu.VMEM((1, H), _f32),
        ],
        compiler_params=pltpu.CompilerParams(
            dimension_semantics=("arbitrary",)),
    )


def _tcb_body(agg_ref, deg_ref, x_ref, pool_ref, w_gcn, b_gcn, g_gcn, bn_gcn,
              wp1a, wp1b, bp1, z_out, hp1_out):
    a = agg_ref[...] * lax.rsqrt(jnp.maximum(deg_ref[...], 1.0))
    t = jnp.dot(a, w_gcn[...], preferred_element_type=_f32) + b_gcn[...]
    g1 = _ln(t, g_gcn[...], bn_gcn[...])
    xb = x_ref[...]
    loc = g1 - xb
    glo = pool_ref[...] - xb
    z_out[...] = jnp.concatenate([loc, glo], axis=1)
    hp = (jnp.dot(loc, wp1a[...], preferred_element_type=_f32)
          + jnp.dot(glo, wp1b[...], preferred_element_type=_f32) + bp1[...])
    hp1_out[...] = jnp.maximum(hp, 0.0)


@functools.lru_cache(maxsize=None)
def _tcb():
    bs = pl.BlockSpec
    return pl.pallas_call(
        _tcb_body,
        grid=(GRID,),
        in_specs=[
            bs((BN, H), lambda i: (i, 0)),
            bs((BN, 1), lambda i: (i, 0)),
            bs((BN, H), lambda i: (i, 0)),
            bs((1, H), lambda i: (0, 0)),
            bs((H, H), lambda i: (0, 0)),
            bs((1, H), lambda i: (0, 0)),
            bs((1, H), lambda i: (0, 0)),
            bs((1, H), lambda i: (0, 0)),
            bs((H, H2), lambda i: (0, 0)),
            bs((H, H2), lambda i: (0, 0)),
            bs((1, H2), lambda i: (0, 0)),
        ],
        out_specs=[
            bs((BN, H2), lambda i: (i, 0)),
            bs((BN, H2), lambda i: (i, 0)),
        ],
        out_shape=[
            jax.ShapeDtypeStruct((N, H2), _f32),
            jax.ShapeDtypeStruct((N, H2), _f32),
        ],
        compiler_params=pltpu.CompilerParams(
            dimension_semantics=("arbitrary",)),
    )


def _tcc1_body(hh_ref, n_ref, ws, wn, bsb, g_ln, b_ln, wp2, bp2,
               hh1_out, hp2_out):
    o = (jnp.dot(hh_ref[...], ws[...], preferred_element_type=_f32)
         + jnp.dot(n_ref[...], wn[...], preferred_element_type=_f32)
         + bsb[...])
    r = jnp.maximum(_ln(o, g_ln[...], b_ln[...]), 0.0)
    hh1_out[...] = r
    hp = jnp.dot(r, wp2[...], preferred_element_type=_f32) + bp2[...]
    hp2_out[...] = jnp.maximum(hp, 0.0)


@functools.lru_cache(maxsize=None)
def _tcc1():
    bs = pl.BlockSpec
    return pl.pallas_call(
        _tcc1_body,
        grid=(GRID,),
        in_specs=[
            bs((BN, H2), lambda i: (i, 0)),
            bs((BN, H2), lambda i: (i, 0)),
            bs((H2, H2), lambda i: (0, 0)),
            bs((H2, H2), lambda i: (0, 0)),
            bs((1, H2), lambda i: (0, 0)),
            bs((1, H2), lambda i: (0, 0)),
            bs((1, H2), lambda i: (0, 0)),
            bs((H2, H2), lambda i: (0, 0)),
            bs((1, H2), lambda i: (0, 0)),
        ],
        out_specs=[
            bs((BN, H2), lambda i: (i, 0)),
            bs((BN, H2), lambda i: (i, 0)),
        ],
        out_shape=[
            jax.ShapeDtypeStruct((N, H2), _f32),
            jax.ShapeDtypeStruct((N, H2), _f32),
        ],
        compiler_params=pltpu.CompilerParams(
            dimension_semantics=("arbitrary",)),
    )


def _tcc2_body(hh_ref, n_ref, ws, wn, bsb, g_ln, b_ln, wo, bo, score_out):
    o = (jnp.dot(hh_ref[...], ws[...], preferred_element_type=_f32)
         + jnp.dot(n_ref[...], wn[...], preferred_element_type=_f32)
         + bsb[...])
    r = jnp.maximum(_ln(o, g_ln[...], b_ln[...]), 0.0)
    score_out[...] = jnp.dot(r, wo[...], preferred_element_type=_f32) + bo[...]


@functools.lru_cache(maxsize=None)
def _tcc2():
    bs = pl.BlockSpec
    return pl.pallas_call(
        _tcc2_body,
        grid=(GRID,),
        in_specs=[
            bs((BN, H2), lambda i: (i, 0)),
            bs((BN, H2), lambda i: (i, 0)),
            bs((H2, H2), lambda i: (0, 0)),
            bs((H2, H2), lambda i: (0, 0)),
            bs((1, H2), lambda i: (0, 0)),
            bs((1, H2), lambda i: (0, 0)),
            bs((1, H2), lambda i: (0, 0)),
            bs((H2, 8), lambda i: (0, 0)),
            bs((1, 8), lambda i: (0, 0)),
        ],
        out_specs=[bs((BN, 8), lambda i: (i, 0))],
        out_shape=[jax.ShapeDtypeStruct((N, 8), _f32)],
        compiler_params=pltpu.CompilerParams(
            dimension_semantics=("arbitrary",)),
    )


def kernel(h, params, edge_index):
    p = params
    src = edge_index[0]
    dst = edge_index[1]

    npad = E_PAD - E
    dead = (jnp.arange(npad, dtype=_i32) % 96) + DEAD
    src_k1 = jnp.concatenate([src, dead]).reshape(ROWS, 128)
    dst_k1 = jnp.concatenate([dst, dead]).reshape(ROWS, 128)
    src_k3 = jnp.concatenate([src, jnp.zeros((npad,), _i32)]).reshape(ROWS, 128)

    deg_out, deg_in = _k1()(src_k1, dst_k1)
    hists = _k2a()(dst_k1)
    perm_comb, offsets = _k2b()(dst_k1, src_k3, hists)

    r2 = lambda a: a.reshape(1, -1)
    x, xs, pool = _tca()(
        h, deg_out.reshape(N, 1),
        p['W_in'], r2(p['b_in']), p['W_t1'], r2(p['b_t1']),
        p['W_t2'], r2(p['b_t2']), p['W_gate'], r2(p['b_gate']))

    agg = _seg_kernel(H, False)(xs, perm_comb, offsets)[:N]

    l1, l2 = p['layers'][0], p['layers'][1]
    z, hp1 = _tcb()(
        agg, deg_in.reshape(N, 1), x, pool,
        p['W_gcn'], r2(p['b_gcn']), r2(p['ln_gcn_g']), r2(p['ln_gcn_b']),
        l1['Wp'][:H], l1['Wp'][H:], r2(l1['bp']))

    n1 = _seg_kernel(H2, True)(hp1, perm_comb, offsets)[:N]
    hh1, hp2 = _tcc1()(
        z, n1, l1['Ws'], l1['Wn'], r2(l1['bs']),
        r2(l1['ln_g']), r2(l1['ln_b']), l2['Wp'], r2(l2['bp']))

    n2 = _seg_kernel(H2, True)(hp2, perm_comb, offsets)[:N]
    wo = jnp.pad(p['W_out'], ((0, 0), (0, 8 - OUT)))
    bo = jnp.pad(p['b_out'], (0, 8 - OUT)).reshape(1, 8)
    (score8,) = _tcc2()(
        hh1, n2, l2['Ws'], l2['Wn'], r2(l2['bs']),
        r2(l2['ln_g']), r2(l2['ln_b']), wo, bo)

    return score8[:, :OUT], z
